# Initial kernel scaffold; baseline (speedup 1.0000x reference)
import jax
import jax.numpy as jnp
from jax.experimental import pallas as pl


def kernel(cf0, cf1, cf2, cf3, cf4, cf5, cf6, cf7, cf8, cf9, cf10, cf11, cf12, sf0, sf1, sf2, sf3, sf4, sf5, sf6, sf7, sf8, sf9, sf10, sf11, sf12, sf13, sf14, sf15, sf16, sf17, sf18, sf19, sf20, sf21, sf22, sf23, sf24, sf25, W1, b1, W2, b2, emb0, emb1, emb2, emb3, emb4, emb5, emb6, emb7, emb8, emb9, emb10, emb11, emb12, emb13, emb14, emb15, emb16, emb17, emb18, emb19, emb20, emb21, emb22, emb23, emb24, emb25):
    raise NotImplementedError("write your pallas kernel here")


# speedup vs baseline: 5.0177x; 5.0177x over previous
import functools

import jax
import jax.numpy as jnp
from jax import lax
from jax.experimental import pallas as pl
from jax.experimental.pallas import tpu as pltpu
from jax.experimental.pallas import tpu_sc as plsc

B = 16384
N_CF = 13
N_SF = 26
VOCAB = 33
D = 64

_NC = 2
_NS = 16
_NW = _NC * _NS
_CHUNK = B // _NW


def _mlp_body(cf_ref, w1_ref, b1_ref, w2_ref, b2_ref, out_ref):
    x = cf_ref[...]
    x = jnp.where(jnp.isnan(x), 0.0, x)
    h = jnp.dot(x, w1_ref[...], preferred_element_type=jnp.float32)
    h = jnp.maximum(h + b1_ref[...], 0.0)
    h = jnp.dot(h, w2_ref[...], preferred_element_type=jnp.float32)
    h = jnp.maximum(h + b2_ref[...], 0.0)
    out_ref[...] = h


def _mlp(cfm, W1, b1, W2, b2):
    bs = 2048
    return pl.pallas_call(
        _mlp_body,
        grid=(B // bs,),
        in_specs=[
            pl.BlockSpec((bs, N_CF), lambda i: (i, 0)),
            pl.BlockSpec((N_CF, 2 * N_CF), lambda i: (0, 0)),
            pl.BlockSpec((1, 2 * N_CF), lambda i: (0, 0)),
            pl.BlockSpec((2 * N_CF, D), lambda i: (0, 0)),
            pl.BlockSpec((1, D), lambda i: (0, 0)),
        ],
        out_specs=pl.BlockSpec((bs, D), lambda i: (i, 0)),
        out_shape=jax.ShapeDtypeStruct((B, D), jnp.float32),
    )(cfm, W1.reshape(N_CF, 2 * N_CF), b1.reshape(1, 2 * N_CF),
      W2.reshape(2 * N_CF, D), b2.reshape(1, D))


def _emb_accumulate(h, sfs, embs):
    mesh = plsc.VectorSubcoreMesh(core_axis_name="c", subcore_axis_name="s")

    @functools.partial(
        pl.kernel,
        mesh=mesh,
        compiler_params=pltpu.CompilerParams(use_tc_tiling_on_sc=False),
        out_type=jax.ShapeDtypeStruct((B, D), jnp.float32),
        scratch_types=[
            pltpu.VMEM((_CHUNK, D), jnp.float32),
            pltpu.VMEM((_CHUNK,), jnp.int32),
            pltpu.VMEM((_CHUNK,), jnp.int32),
            pltpu.SemaphoreType.DMA,
            pltpu.SemaphoreType.DMA,
        ],
    )
    def k(*refs):
        h_hbm = refs[0]
        sf_refs = refs[1:1 + N_SF]
        emb_refs = refs[1 + N_SF:1 + 2 * N_SF]
        out_hbm = refs[1 + 2 * N_SF]
        acc_v, idx_a, idx_b, sem_g, sem_i = refs[2 + 2 * N_SF:]
        idx_bufs = (idx_a, idx_b)

        wid = lax.axis_index("s") * _NC + lax.axis_index("c")
        base = wid * _CHUNK
        rows = pl.ds(base, _CHUNK)

        cp_h = pltpu.async_copy(h_hbm.at[rows], acc_v, sem_i)
        pltpu.sync_copy(sf_refs[0].at[rows], idx_bufs[0])
        cp_h.wait()
        for i in range(N_SF):
            if i + 1 < N_SF:
                cp_idx = pltpu.async_copy(
                    sf_refs[i + 1].at[rows], idx_bufs[(i + 1) % 2], sem_i)
            g = pltpu.async_copy(
                emb_refs[i].at[idx_bufs[i % 2]], acc_v, sem_g, add=True)
            g.wait()
            if i + 1 < N_SF:
                cp_idx.wait()
        pltpu.sync_copy(acc_v, out_hbm.at[rows])

    return k(h, *sfs, *embs)


def kernel(cf0, cf1, cf2, cf3, cf4, cf5, cf6, cf7, cf8, cf9, cf10, cf11,
           cf12, sf0, sf1, sf2, sf3, sf4, sf5, sf6, sf7, sf8, sf9, sf10,
           sf11, sf12, sf13, sf14, sf15, sf16, sf17, sf18, sf19, sf20,
           sf21, sf22, sf23, sf24, sf25, W1, b1, W2, b2, emb0, emb1, emb2,
           emb3, emb4, emb5, emb6, emb7, emb8, emb9, emb10, emb11, emb12,
           emb13, emb14, emb15, emb16, emb17, emb18, emb19, emb20, emb21,
           emb22, emb23, emb24, emb25):
    cfs = [cf0, cf1, cf2, cf3, cf4, cf5, cf6, cf7, cf8, cf9, cf10, cf11,
           cf12]
    sfs = [sf0, sf1, sf2, sf3, sf4, sf5, sf6, sf7, sf8, sf9, sf10, sf11,
           sf12, sf13, sf14, sf15, sf16, sf17, sf18, sf19, sf20, sf21,
           sf22, sf23, sf24, sf25]
    embs = [emb0, emb1, emb2, emb3, emb4, emb5, emb6, emb7, emb8, emb9,
            emb10, emb11, emb12, emb13, emb14, emb15, emb16, emb17, emb18,
            emb19, emb20, emb21, emb22, emb23, emb24, emb25]
    cfm = jnp.stack(cfs, axis=1)
    h = _mlp(cfm, W1, b1, W2, b2)
    return _emb_accumulate(h, sfs, embs)


# speedup vs baseline: 15.6691x; 3.1228x over previous
import functools

import jax
import jax.numpy as jnp
from jax import lax
from jax.experimental import pallas as pl
from jax.experimental.pallas import tpu as pltpu
from jax.experimental.pallas import tpu_sc as plsc

B = 16384
N_CF = 13
N_SF = 26
VOCAB = 33
D = 64

_NC = 2
_NS = 16
_NW = _NC * _NS
_CHUNK = B // _NW


def _mlp_body(cf_ref, w1_ref, b1_ref, w2_ref, b2_ref, out_ref):
    x = cf_ref[...]
    x = jnp.where(jnp.isnan(x), 0.0, x)
    h = jnp.dot(x, w1_ref[...], preferred_element_type=jnp.float32)
    h = jnp.maximum(h + b1_ref[...], 0.0)
    h = jnp.dot(h, w2_ref[...], preferred_element_type=jnp.float32)
    h = jnp.maximum(h + b2_ref[...], 0.0)
    out_ref[...] = h


def _mlp(cfm, W1, b1, W2, b2):
    bs = 2048
    return pl.pallas_call(
        _mlp_body,
        grid=(B // bs,),
        in_specs=[
            pl.BlockSpec((bs, N_CF), lambda i: (i, 0)),
            pl.BlockSpec((N_CF, 2 * N_CF), lambda i: (0, 0)),
            pl.BlockSpec((1, 2 * N_CF), lambda i: (0, 0)),
            pl.BlockSpec((2 * N_CF, D), lambda i: (0, 0)),
            pl.BlockSpec((1, D), lambda i: (0, 0)),
        ],
        out_specs=pl.BlockSpec((bs, D), lambda i: (i, 0)),
        out_shape=jax.ShapeDtypeStruct((B, D), jnp.float32),
    )(cfm, W1.reshape(N_CF, 2 * N_CF), b1.reshape(1, 2 * N_CF),
      W2.reshape(2 * N_CF, D), b2.reshape(1, D))


def _emb_accumulate(h, sfs, table):
    mesh = plsc.VectorSubcoreMesh(core_axis_name="c", subcore_axis_name="s")

    @functools.partial(
        pl.kernel,
        mesh=mesh,
        compiler_params=pltpu.CompilerParams(use_tc_tiling_on_sc=False),
        out_type=jax.ShapeDtypeStruct((B, D), jnp.float32),
        scratch_types=[
            pltpu.VMEM_SHARED((N_SF * VOCAB, D), jnp.float32),
            pltpu.VMEM((_CHUNK, D), jnp.float32),
            pltpu.VMEM((_CHUNK,), jnp.int32),
            pltpu.VMEM((_CHUNK,), jnp.int32),
            pltpu.VMEM((_CHUNK,), jnp.int32),
            pltpu.SemaphoreType.DMA,
            pltpu.SemaphoreType.DMA,
        ],
    )
    def k(*refs):
        h_hbm = refs[0]
        sf_refs = refs[1:1 + N_SF]
        t_hbm = refs[1 + N_SF]
        out_hbm = refs[2 + N_SF]
        tab_sp, acc_v, idx_a, idx_b, idxo_v, sem_g, sem_i = refs[3 + N_SF:]
        idx_bufs = (idx_a, idx_b)

        sid = lax.axis_index("s")
        wid = sid * _NC + lax.axis_index("c")
        base = wid * _CHUNK
        rows = pl.ds(base, _CHUNK)

        cp_h = pltpu.async_copy(h_hbm.at[rows], acc_v, sem_i)
        @pl.when(sid == 0)
        def _():
            pltpu.sync_copy(t_hbm, tab_sp)
        pltpu.sync_copy(sf_refs[0].at[rows], idx_bufs[0])
        cp_h.wait()
        plsc.subcore_barrier()
        for i in range(N_SF):
            if i + 1 < N_SF:
                cp_idx = pltpu.async_copy(
                    sf_refs[i + 1].at[rows], idx_bufs[(i + 1) % 2], sem_i)
            src = idx_bufs[i % 2]
            for o in range(0, _CHUNK, 16):
                sl = pl.ds(o, 16)
                idxo_v[sl] = src[sl] + (VOCAB * i)
            g = pltpu.async_copy(tab_sp.at[idxo_v], acc_v, sem_g, add=True)
            g.wait()
            if i + 1 < N_SF:
                cp_idx.wait()
        pltpu.sync_copy(acc_v, out_hbm.at[rows])

    return k(h, *sfs, table)


def kernel(cf0, cf1, cf2, cf3, cf4, cf5, cf6, cf7, cf8, cf9, cf10, cf11,
           cf12, sf0, sf1, sf2, sf3, sf4, sf5, sf6, sf7, sf8, sf9, sf10,
           sf11, sf12, sf13, sf14, sf15, sf16, sf17, sf18, sf19, sf20,
           sf21, sf22, sf23, sf24, sf25, W1, b1, W2, b2, emb0, emb1, emb2,
           emb3, emb4, emb5, emb6, emb7, emb8, emb9, emb10, emb11, emb12,
           emb13, emb14, emb15, emb16, emb17, emb18, emb19, emb20, emb21,
           emb22, emb23, emb24, emb25):
    cfs = [cf0, cf1, cf2, cf3, cf4, cf5, cf6, cf7, cf8, cf9, cf10, cf11,
           cf12]
    sfs = [sf0, sf1, sf2, sf3, sf4, sf5, sf6, sf7, sf8, sf9, sf10, sf11,
           sf12, sf13, sf14, sf15, sf16, sf17, sf18, sf19, sf20, sf21,
           sf22, sf23, sf24, sf25]
    embs = [emb0, emb1, emb2, emb3, emb4, emb5, emb6, emb7, emb8, emb9,
            emb10, emb11, emb12, emb13, emb14, emb15, emb16, emb17, emb18,
            emb19, emb20, emb21, emb22, emb23, emb24, emb25]
    cfm = jnp.stack(cfs, axis=1)
    table = jnp.concatenate(embs, axis=0)
    h = _mlp(cfm, W1, b1, W2, b2)
    return _emb_accumulate(h, sfs, table)
